# baseline (speedup 1.0000x reference)
import jax
import jax.numpy as jnp
from jax import lax
from jax.experimental import pallas as pl
from jax.experimental.pallas import tpu as pltpu
from jax.experimental.pallas import tpu_sc as plsc

N_NODES = 50000
N_PAD = 50048
DIM = 64
HALF = 32
N_EDGES = 800000
G = 3
E_CHUNK = G * 128
CHUNKS = 136
E_PAD = 16 * CHUNKS * E_CHUNK
BATCH = 4096

_MESH = plsc.VectorSubcoreMesh(core_axis_name="c", subcore_axis_name="s")


def _agg_body(sidx, didx, tlo, thi, zeros, out_lo, out_hi,
              shared, rows0, rows1, isem0, isem1, gsem0, gsem1,
              ssem0, ssem1, *ivecs):
    rows = (rows0, rows1)
    siv = (ivecs[0:G], ivecs[G:2 * G])
    dv = ivecs[2 * G:]
    div = (dv[0:G], dv[G:2 * G])
    isem = (isem0, isem1)
    gsem = (gsem0, gsem1)
    ssem = (ssem0, ssem1)
    c = lax.axis_index("c")
    s = lax.axis_index("s")

    z0 = s * (N_PAD // 16)
    pltpu.sync_copy(zeros.at[pl.ds(z0, N_PAD // 16)],
                    shared.at[pl.ds(z0, N_PAD // 16)])
    plsc.subcore_barrier()

    def run(table):
        def idx_start(k, b):
            base = (s * CHUNKS + k) * E_CHUNK
            for j in range(G):
                pltpu.async_copy(sidx.at[pl.ds(base + j * 128, 128)],
                                 siv[b][j], isem[b])
                pltpu.async_copy(didx.at[pl.ds(base + j * 128, 128)],
                                 div[b][j], isem[b])

        def idx_wait(b):
            for j in range(G):
                pltpu.make_async_copy(sidx.at[pl.ds(0, 128)],
                                      siv[b][j], isem[b]).wait()
                pltpu.make_async_copy(didx.at[pl.ds(0, 128)],
                                      div[b][j], isem[b]).wait()

        def gathers(b):
            return [
                pltpu.async_copy(table.at[siv[b][j]],
                                 rows[b].at[pl.ds(j * 128, 128)], gsem[b])
                for j in range(G)
            ]

        def scatters(b):
            for j in range(G):
                pltpu.async_copy(rows[b].at[pl.ds(j * 128, 128)],
                                 shared.at[div[b][j]], ssem[b], add=True)

        def scatter_drain(b):
            for j in range(G):
                pltpu.make_async_copy(rows[b].at[pl.ds(j * 128, 128)],
                                      shared.at[div[b][j]], ssem[b]).wait()

        idx_start(0, 0)

        def pair(g, carry):
            for b in (0, 1):
                k = 2 * g + b
                idx_wait(b)
                handles = gathers(b)
                if b == 0:
                    @pl.when(g > 0)
                    def _():
                        scatter_drain(1)
                else:
                    scatter_drain(0)
                idx_start(k + 1, 1 - b)
                for h in handles:
                    h.wait()
                scatters(b)
            return carry
        lax.fori_loop(0, CHUNKS // 2, pair, 0)
        idx_wait(0)
        scatter_drain(1)

    @pl.when(c == 0)
    def _():
        run(tlo)

    @pl.when(c == 1)
    def _():
        run(thi)

    plsc.subcore_barrier()

    w0 = s * (N_PAD // 16)
    @pl.when(c == 0)
    def _():
        pltpu.sync_copy(shared.at[pl.ds(w0, N_PAD // 16)],
                        out_lo.at[pl.ds(w0, N_PAD // 16)])

    @pl.when(c == 1)
    def _():
        pltpu.sync_copy(shared.at[pl.ds(w0, N_PAD // 16)],
                        out_hi.at[pl.ds(w0, N_PAD // 16)])


_agg = pl.kernel(
    _agg_body,
    out_type=(
        jax.ShapeDtypeStruct((N_PAD, HALF), jnp.float32),
        jax.ShapeDtypeStruct((N_PAD, HALF), jnp.float32),
    ),
    mesh=_MESH,
    compiler_params=pltpu.CompilerParams(use_tc_tiling_on_sc=False),
    scratch_types=[
        pltpu.VMEM_SHARED((N_PAD, HALF), jnp.float32),
        pltpu.VMEM((E_CHUNK, HALF), jnp.float32),
        pltpu.VMEM((E_CHUNK, HALF), jnp.float32),
    ] + [pltpu.SemaphoreType.DMA] * 6
      + [pltpu.VMEM((128,), jnp.int32) for _ in range(4 * G)],
)


def _bg_body(su_lo, su_hi, si_lo, si_hi, uidx, iidx,
             gul, guh, gil, gih, uiv, iiv, b0, b1, b2, b3, sem):
    c = lax.axis_index("c")
    s = lax.axis_index("s")
    w = s * 2 + c
    pltpu.sync_copy(uidx.at[pl.ds(w * 128, 128)], uiv)
    pltpu.sync_copy(iidx.at[pl.ds(w * 128, 128)], iiv)
    handles = [
        pltpu.async_copy(su_lo.at[uiv], b0, sem),
        pltpu.async_copy(su_hi.at[uiv], b1, sem),
        pltpu.async_copy(si_lo.at[iiv], b2, sem),
        pltpu.async_copy(si_hi.at[iiv], b3, sem),
    ]
    for h in handles:
        h.wait()
    pltpu.sync_copy(b0, gul.at[pl.ds(w * 128, 128)])
    pltpu.sync_copy(b1, guh.at[pl.ds(w * 128, 128)])
    pltpu.sync_copy(b2, gil.at[pl.ds(w * 128, 128)])
    pltpu.sync_copy(b3, gih.at[pl.ds(w * 128, 128)])


_batch_gather = pl.kernel(
    _bg_body,
    out_type=tuple(
        jax.ShapeDtypeStruct((BATCH, HALF), jnp.float32) for _ in range(4)),
    mesh=_MESH,
    compiler_params=pltpu.CompilerParams(use_tc_tiling_on_sc=False),
    scratch_types=[
        pltpu.VMEM((128,), jnp.int32),
        pltpu.VMEM((128,), jnp.int32),
        pltpu.VMEM((128, HALF), jnp.float32),
        pltpu.VMEM((128, HALF), jnp.float32),
        pltpu.VMEM((128, HALF), jnp.float32),
        pltpu.VMEM((128, HALF), jnp.float32),
        pltpu.SemaphoreType.DMA,
    ],
)


def _norm_kernel(ualo, uahi, uslo, ushi, ialo, iahi, islo, ishi,
                 uelo, uehi, uolo, uohi, ielo, iehi, iolo, iohi):
    def one(alo, ahi, slo, shi, elo, ehi, olo, ohi):
        lo = alo[...]
        hi = ahi[...]
        n2 = (jnp.sum(lo * lo, axis=1, keepdims=True)
              + jnp.sum(hi * hi, axis=1, keepdims=True))
        inv = 1.0 / jnp.maximum(jnp.sqrt(n2), 1e-12)
        nlo = lo * inv
        nhi = hi * inv
        elo[...] = nlo
        ehi[...] = nhi
        olo[...] = slo[...] + nlo
        ohi[...] = shi[...] + nhi
    one(ualo, uahi, uslo, ushi, uelo, uehi, uolo, uohi)
    one(ialo, iahi, islo, ishi, ielo, iehi, iolo, iohi)


_NORM_BLOCK = N_PAD // 16
_norm = pl.pallas_call(
    _norm_kernel,
    grid=(N_PAD // _NORM_BLOCK,),
    in_specs=[pl.BlockSpec((_NORM_BLOCK, HALF), lambda i: (i, 0))] * 8,
    out_specs=[pl.BlockSpec((_NORM_BLOCK, HALF), lambda i: (i, 0))] * 8,
    out_shape=tuple(
        jax.ShapeDtypeStruct((N_PAD, HALF), jnp.float32) for _ in range(8)),
)


def _dot_kernel(ul, uh, il, ih, o):
    o[...] = jnp.sum(ul[...] * il[...] + uh[...] * ih[...],
                     axis=1, keepdims=True) * 0.0625


_dot = pl.pallas_call(
    _dot_kernel,
    out_shape=jax.ShapeDtypeStruct((BATCH, 1), jnp.float32),
)


def kernel(user_indices, item_indices, edge_index, user_table, item_table):
    i32 = jnp.int32
    u_nodes = edge_index[0].astype(i32)
    i_nodes = edge_index[1].astype(i32)
    pad = E_PAD - N_EDGES
    dummy_dst = jnp.full((pad,), N_NODES, dtype=i32)
    zero_src = jnp.zeros((pad,), dtype=i32)

    tail = jnp.zeros((2 * E_CHUNK,), i32)
    ua_src = jnp.concatenate([i_nodes, zero_src, tail])
    ua_dst = jnp.concatenate([u_nodes, dummy_dst, tail])
    ia_src = jnp.concatenate([u_nodes, zero_src, tail])
    ia_dst = jnp.concatenate([i_nodes, dummy_dst, tail])
    zeros = jnp.zeros((N_PAD, HALF), jnp.float32)

    rpad = jnp.zeros((N_PAD - N_NODES, HALF), jnp.float32)
    ut_lo = jnp.concatenate([user_table[:, :HALF], rpad])
    ut_hi = jnp.concatenate([user_table[:, HALF:], rpad])
    it_lo = jnp.concatenate([item_table[:, :HALF], rpad])
    it_hi = jnp.concatenate([item_table[:, HALF:], rpad])
    sum_u_lo, sum_u_hi = ut_lo, ut_hi
    sum_i_lo, sum_i_hi = it_lo, it_hi
    ue_lo, ue_hi = ut_lo, ut_hi
    ie_lo, ie_hi = it_lo, it_hi

    for _ in range(3):
        ua_lo, ua_hi = _agg(ua_src, ua_dst, ie_lo, ie_hi, zeros)
        ia_lo, ia_hi = _agg(ia_src, ia_dst, ue_lo, ue_hi, zeros)
        (ue_lo, ue_hi, sum_u_lo, sum_u_hi,
         ie_lo, ie_hi, sum_i_lo, sum_i_hi) = _norm(
            ua_lo, ua_hi, sum_u_lo, sum_u_hi,
            ia_lo, ia_hi, sum_i_lo, sum_i_hi)

    uidx = user_indices.astype(i32)
    iidx = item_indices.astype(i32)
    gul, guh, gil, gih = _batch_gather(sum_u_lo, sum_u_hi,
                                       sum_i_lo, sum_i_hi, uidx, iidx)
    return _dot(gul, guh, gil, gih).reshape(BATCH)


# speedup vs baseline: 1.3736x; 1.3736x over previous
import jax
import jax.numpy as jnp
from jax import lax
from jax.experimental import pallas as pl
from jax.experimental.pallas import tpu as pltpu
from jax.experimental.pallas import tpu_sc as plsc

N_NODES = 50000
N_PAD = 50048
DIM = 64
HALF = 32
N_EDGES = 800000
G = 3
E_CHUNK = G * 128
CHUNKS = 134
E_PAD = 16 * CHUNKS * E_CHUNK
BATCH = 4096

_MESH = plsc.VectorSubcoreMesh(core_axis_name="c", subcore_axis_name="s")


def _agg_body(sidx, didx, tlo, thi, zeros, out_lo, out_hi,
              shared, rows0, rows1, isem0, isem1, gsem0, gsem1,
              ssem0, ssem1, *ivecs):
    rows = (rows0, rows1)
    siv = (ivecs[0:G], ivecs[G:2 * G])
    dv = ivecs[2 * G:]
    div = (dv[0:G], dv[G:2 * G])
    isem = (isem0, isem1)
    gsem = (gsem0, gsem1)
    ssem = (ssem0, ssem1)
    c = lax.axis_index("c")
    s = lax.axis_index("s")

    z0 = s * (N_PAD // 16)
    pltpu.sync_copy(zeros.at[pl.ds(z0, N_PAD // 16)],
                    shared.at[pl.ds(z0, N_PAD // 16)])
    plsc.subcore_barrier()

    def run(table):
        def idx_start(k, b):
            base = (s * CHUNKS + k) * E_CHUNK
            for j in range(G):
                pltpu.async_copy(sidx.at[pl.ds(base + j * 128, 128)],
                                 siv[b][j], isem[b])
                pltpu.async_copy(didx.at[pl.ds(base + j * 128, 128)],
                                 div[b][j], isem[b])

        def idx_wait(b):
            for j in range(G):
                pltpu.make_async_copy(sidx.at[pl.ds(0, 128)],
                                      siv[b][j], isem[b]).wait()
                pltpu.make_async_copy(didx.at[pl.ds(0, 128)],
                                      div[b][j], isem[b]).wait()

        def gathers(b):
            return [
                pltpu.async_copy(table.at[siv[b][j]],
                                 rows[b].at[pl.ds(j * 128, 128)], gsem[b])
                for j in range(G)
            ]

        def scatters(b):
            for j in range(G):
                pltpu.async_copy(rows[b].at[pl.ds(j * 128, 128)],
                                 shared.at[div[b][j]], ssem[b], add=True)

        def scatter_drain(b):
            for j in range(G):
                pltpu.make_async_copy(rows[b].at[pl.ds(j * 128, 128)],
                                      shared.at[div[b][j]], ssem[b]).wait()

        idx_start(0, 0)

        def pair(g, carry):
            for b in (0, 1):
                k = 2 * g + b
                idx_wait(b)
                handles = gathers(b)
                if b == 0:
                    @pl.when(g > 0)
                    def _():
                        scatter_drain(1)
                else:
                    scatter_drain(0)
                idx_start(k + 1, 1 - b)
                for h in handles:
                    h.wait()
                scatters(b)
            return carry
        lax.fori_loop(0, CHUNKS // 2, pair, 0)
        idx_wait(0)
        scatter_drain(1)

    @pl.when(c == 0)
    def _():
        run(tlo)

    @pl.when(c == 1)
    def _():
        run(thi)

    plsc.subcore_barrier()

    w0 = s * (N_PAD // 16)
    @pl.when(c == 0)
    def _():
        pltpu.sync_copy(shared.at[pl.ds(w0, N_PAD // 16)],
                        out_lo.at[pl.ds(w0, N_PAD // 16)])

    @pl.when(c == 1)
    def _():
        pltpu.sync_copy(shared.at[pl.ds(w0, N_PAD // 16)],
                        out_hi.at[pl.ds(w0, N_PAD // 16)])


_agg = pl.kernel(
    _agg_body,
    out_type=(
        jax.ShapeDtypeStruct((N_PAD, HALF), jnp.float32),
        jax.ShapeDtypeStruct((N_PAD, HALF), jnp.float32),
    ),
    mesh=_MESH,
    compiler_params=pltpu.CompilerParams(use_tc_tiling_on_sc=False),
    scratch_types=[
        pltpu.VMEM_SHARED((N_PAD, HALF), jnp.float32),
        pltpu.VMEM((E_CHUNK, HALF), jnp.float32),
        pltpu.VMEM((E_CHUNK, HALF), jnp.float32),
    ] + [pltpu.SemaphoreType.DMA] * 6
      + [pltpu.VMEM((128,), jnp.int32) for _ in range(4 * G)],
)


def _bg_body(su_lo, su_hi, si_lo, si_hi, uidx, iidx,
             gul, guh, gil, gih, uiv, iiv, b0, b1, b2, b3, sem):
    c = lax.axis_index("c")
    s = lax.axis_index("s")
    w = s * 2 + c
    pltpu.sync_copy(uidx.at[pl.ds(w * 128, 128)], uiv)
    pltpu.sync_copy(iidx.at[pl.ds(w * 128, 128)], iiv)
    handles = [
        pltpu.async_copy(su_lo.at[uiv], b0, sem),
        pltpu.async_copy(su_hi.at[uiv], b1, sem),
        pltpu.async_copy(si_lo.at[iiv], b2, sem),
        pltpu.async_copy(si_hi.at[iiv], b3, sem),
    ]
    for h in handles:
        h.wait()
    pltpu.sync_copy(b0, gul.at[pl.ds(w * 128, 128)])
    pltpu.sync_copy(b1, guh.at[pl.ds(w * 128, 128)])
    pltpu.sync_copy(b2, gil.at[pl.ds(w * 128, 128)])
    pltpu.sync_copy(b3, gih.at[pl.ds(w * 128, 128)])


_batch_gather = pl.kernel(
    _bg_body,
    out_type=tuple(
        jax.ShapeDtypeStruct((BATCH, HALF), jnp.float32) for _ in range(4)),
    mesh=_MESH,
    compiler_params=pltpu.CompilerParams(use_tc_tiling_on_sc=False),
    scratch_types=[
        pltpu.VMEM((128,), jnp.int32),
        pltpu.VMEM((128,), jnp.int32),
        pltpu.VMEM((128, HALF), jnp.float32),
        pltpu.VMEM((128, HALF), jnp.float32),
        pltpu.VMEM((128, HALF), jnp.float32),
        pltpu.VMEM((128, HALF), jnp.float32),
        pltpu.SemaphoreType.DMA,
    ],
)


def _norm_kernel(alo, ahi, slo, shi, elo, ehi, olo, ohi):
    lo = alo[...]
    hi = ahi[...]
    n2 = (jnp.sum(lo * lo, axis=1, keepdims=True)
          + jnp.sum(hi * hi, axis=1, keepdims=True))
    inv = 1.0 / jnp.maximum(jnp.sqrt(n2), 1e-12)
    nlo = lo * inv
    nhi = hi * inv
    elo[...] = nlo
    ehi[...] = nhi
    olo[...] = slo[...] + nlo
    ohi[...] = shi[...] + nhi


_NORM_BLOCK = N_PAD // 16
_norm = pl.pallas_call(
    _norm_kernel,
    grid=(N_PAD // _NORM_BLOCK,),
    in_specs=[pl.BlockSpec((_NORM_BLOCK, HALF), lambda i: (i, 0))] * 4,
    out_specs=[pl.BlockSpec((_NORM_BLOCK, HALF), lambda i: (i, 0))] * 4,
    out_shape=tuple(
        jax.ShapeDtypeStruct((N_PAD, HALF), jnp.float32) for _ in range(4)),
)


def _dot_kernel(ul, uh, il, ih, o):
    o[...] = jnp.sum(ul[...] * il[...] + uh[...] * ih[...],
                     axis=1, keepdims=True) * 0.0625


_dot = pl.pallas_call(
    _dot_kernel,
    out_shape=jax.ShapeDtypeStruct((BATCH, 1), jnp.float32),
)


def kernel(user_indices, item_indices, edge_index, user_table, item_table):
    i32 = jnp.int32
    u_nodes = edge_index[0].astype(i32)
    i_nodes = edge_index[1].astype(i32)
    pad = E_PAD - N_EDGES
    dummy_dst = jnp.full((pad,), N_NODES, dtype=i32)
    zero_src = jnp.zeros((pad,), dtype=i32)

    tail = jnp.zeros((2 * E_CHUNK,), i32)
    ua_src = jnp.concatenate([i_nodes, zero_src, tail])
    ua_dst = jnp.concatenate([u_nodes, dummy_dst, tail])
    ia_src = jnp.concatenate([u_nodes, zero_src, tail])
    ia_dst = jnp.concatenate([i_nodes, dummy_dst, tail])
    zeros = jnp.zeros((N_PAD, HALF), jnp.float32)

    rpad = jnp.zeros((N_PAD - N_NODES, HALF), jnp.float32)
    ut_lo = jnp.concatenate([user_table[:, :HALF], rpad])
    ut_hi = jnp.concatenate([user_table[:, HALF:], rpad])
    it_lo = jnp.concatenate([item_table[:, :HALF], rpad])
    it_hi = jnp.concatenate([item_table[:, HALF:], rpad])
    sum_u_lo, sum_u_hi = ut_lo, ut_hi
    sum_i_lo, sum_i_hi = it_lo, it_hi
    ue_lo, ue_hi = ut_lo, ut_hi
    ie_lo, ie_hi = it_lo, it_hi

    for _ in range(3):
        ua_lo, ua_hi = _agg(ua_src, ua_dst, ie_lo, ie_hi, zeros)
        ia_lo, ia_hi = _agg(ia_src, ia_dst, ue_lo, ue_hi, zeros)
        ue_lo, ue_hi, sum_u_lo, sum_u_hi = _norm(ua_lo, ua_hi,
                                                 sum_u_lo, sum_u_hi)
        ie_lo, ie_hi, sum_i_lo, sum_i_hi = _norm(ia_lo, ia_hi,
                                                 sum_i_lo, sum_i_hi)

    uidx = user_indices.astype(i32)
    iidx = item_indices.astype(i32)
    gul, guh, gil, gih = _batch_gather(sum_u_lo, sum_u_hi,
                                       sum_i_lo, sum_i_hi, uidx, iidx)
    return _dot(gul, guh, gil, gih).reshape(BATCH)


# speedup vs baseline: 1.3748x; 1.0009x over previous
import jax
import jax.numpy as jnp
from jax import lax
from jax.experimental import pallas as pl
from jax.experimental.pallas import tpu as pltpu
from jax.experimental.pallas import tpu_sc as plsc

N_NODES = 50000
N_PAD = 50048
DIM = 64
HALF = 32
N_EDGES = 800000
G = 3
E_CHUNK = G * 128
CHUNKS = 134
E_PAD = 16 * CHUNKS * E_CHUNK
BATCH = 4096

_MESH = plsc.VectorSubcoreMesh(core_axis_name="c", subcore_axis_name="s")


def _agg_body(sidx, didx, tlo, thi, zeros, out_lo, out_hi,
              shared, rows0, rows1, isem0, isem1, gsem0, gsem1,
              ssem0, ssem1, *ivecs):
    rows = (rows0, rows1)
    siv = (ivecs[0:G], ivecs[G:2 * G])
    dv = ivecs[2 * G:]
    div = (dv[0:G], dv[G:2 * G])
    isem = (isem0, isem1)
    gsem = (gsem0, gsem1)
    ssem = (ssem0, ssem1)
    c = lax.axis_index("c")
    s = lax.axis_index("s")

    z0 = s * (N_PAD // 16)
    pltpu.sync_copy(zeros.at[pl.ds(z0, N_PAD // 16)],
                    shared.at[pl.ds(z0, N_PAD // 16)])
    plsc.subcore_barrier()

    def run(table):
        def idx_start(k, b):
            base = (s * CHUNKS + k) * E_CHUNK
            for j in range(G):
                pltpu.async_copy(sidx.at[pl.ds(base + j * 128, 128)],
                                 siv[b][j], isem[b])
                pltpu.async_copy(didx.at[pl.ds(base + j * 128, 128)],
                                 div[b][j], isem[b])

        def idx_wait(b):
            for j in range(G):
                pltpu.make_async_copy(sidx.at[pl.ds(0, 128)],
                                      siv[b][j], isem[b]).wait()
                pltpu.make_async_copy(didx.at[pl.ds(0, 128)],
                                      div[b][j], isem[b]).wait()

        def gathers(b):
            return [
                pltpu.async_copy(table.at[siv[b][j]],
                                 rows[b].at[pl.ds(j * 128, 128)], gsem[b])
                for j in range(G)
            ]

        def scatters(b):
            for j in range(G):
                pltpu.async_copy(rows[b].at[pl.ds(j * 128, 128)],
                                 shared.at[div[b][j]], ssem[b], add=True)

        def scatter_drain(b):
            for j in range(G):
                pltpu.make_async_copy(rows[b].at[pl.ds(j * 128, 128)],
                                      shared.at[div[b][j]], ssem[b]).wait()

        idx_start(0, 0)

        def pair(g, carry):
            for b in (0, 1):
                k = 2 * g + b
                idx_wait(b)
                handles = gathers(b)
                if b == 0:
                    @pl.when(g > 0)
                    def _():
                        scatter_drain(1)
                else:
                    scatter_drain(0)
                idx_start(k + 1, 1 - b)
                for h in handles:
                    h.wait()
                scatters(b)
            return carry
        lax.fori_loop(0, CHUNKS // 2, pair, 0)
        idx_wait(0)
        scatter_drain(1)

    @pl.when(c == 0)
    def _():
        run(tlo)

    @pl.when(c == 1)
    def _():
        run(thi)

    plsc.subcore_barrier()

    w0 = s * (N_PAD // 16)
    @pl.when(c == 0)
    def _():
        pltpu.sync_copy(shared.at[pl.ds(w0, N_PAD // 16)],
                        out_lo.at[pl.ds(w0, N_PAD // 16)])

    @pl.when(c == 1)
    def _():
        pltpu.sync_copy(shared.at[pl.ds(w0, N_PAD // 16)],
                        out_hi.at[pl.ds(w0, N_PAD // 16)])


_agg = pl.kernel(
    _agg_body,
    out_type=(
        jax.ShapeDtypeStruct((N_PAD, HALF), jnp.float32),
        jax.ShapeDtypeStruct((N_PAD, HALF), jnp.float32),
    ),
    mesh=_MESH,
    compiler_params=pltpu.CompilerParams(use_tc_tiling_on_sc=False),
    scratch_types=[
        pltpu.VMEM_SHARED((N_PAD, HALF), jnp.float32),
        pltpu.VMEM((E_CHUNK, HALF), jnp.float32),
        pltpu.VMEM((E_CHUNK, HALF), jnp.float32),
    ] + [pltpu.SemaphoreType.DMA] * 6
      + [pltpu.VMEM((128,), jnp.int32) for _ in range(4 * G)],
)


def _bg_body(su_lo, su_hi, si_lo, si_hi, uidx, iidx,
             gul, guh, gil, gih, uiv, iiv, b0, b1, b2, b3, sem):
    c = lax.axis_index("c")
    s = lax.axis_index("s")
    w = s * 2 + c
    pltpu.sync_copy(uidx.at[pl.ds(w * 128, 128)], uiv)
    pltpu.sync_copy(iidx.at[pl.ds(w * 128, 128)], iiv)
    handles = [
        pltpu.async_copy(su_lo.at[uiv], b0, sem),
        pltpu.async_copy(su_hi.at[uiv], b1, sem),
        pltpu.async_copy(si_lo.at[iiv], b2, sem),
        pltpu.async_copy(si_hi.at[iiv], b3, sem),
    ]
    for h in handles:
        h.wait()
    pltpu.sync_copy(b0, gul.at[pl.ds(w * 128, 128)])
    pltpu.sync_copy(b1, guh.at[pl.ds(w * 128, 128)])
    pltpu.sync_copy(b2, gil.at[pl.ds(w * 128, 128)])
    pltpu.sync_copy(b3, gih.at[pl.ds(w * 128, 128)])


_batch_gather = pl.kernel(
    _bg_body,
    out_type=tuple(
        jax.ShapeDtypeStruct((BATCH, HALF), jnp.float32) for _ in range(4)),
    mesh=_MESH,
    compiler_params=pltpu.CompilerParams(use_tc_tiling_on_sc=False),
    scratch_types=[
        pltpu.VMEM((128,), jnp.int32),
        pltpu.VMEM((128,), jnp.int32),
        pltpu.VMEM((128, HALF), jnp.float32),
        pltpu.VMEM((128, HALF), jnp.float32),
        pltpu.VMEM((128, HALF), jnp.float32),
        pltpu.VMEM((128, HALF), jnp.float32),
        pltpu.SemaphoreType.DMA,
    ],
)


def _norm_kernel(alo, ahi, slo, shi, elo, ehi, olo, ohi):
    lo = alo[...]
    hi = ahi[...]
    n2 = (jnp.sum(lo * lo, axis=1, keepdims=True)
          + jnp.sum(hi * hi, axis=1, keepdims=True))
    inv = 1.0 / jnp.maximum(jnp.sqrt(n2), 1e-12)
    nlo = lo * inv
    nhi = hi * inv
    elo[...] = nlo
    ehi[...] = nhi
    olo[...] = slo[...] + nlo
    ohi[...] = shi[...] + nhi


_NORM_BLOCK = N_PAD // 16
_norm = pl.pallas_call(
    _norm_kernel,
    grid=(N_PAD // _NORM_BLOCK,),
    in_specs=[pl.BlockSpec((_NORM_BLOCK, HALF), lambda i: (i, 0))] * 4,
    out_specs=[pl.BlockSpec((_NORM_BLOCK, HALF), lambda i: (i, 0))] * 4,
    out_shape=tuple(
        jax.ShapeDtypeStruct((N_PAD, HALF), jnp.float32) for _ in range(4)),
)


def _dot_kernel(ul, uh, il, ih, o):
    o[...] = jnp.sum(ul[...] * il[...] + uh[...] * ih[...],
                     axis=1, keepdims=True) * 0.0625


_dot = pl.pallas_call(
    _dot_kernel,
    out_shape=jax.ShapeDtypeStruct((BATCH, 1), jnp.float32),
)


def kernel(user_indices, item_indices, edge_index, user_table, item_table):
    i32 = jnp.int32
    u_nodes = edge_index[0].astype(i32)
    i_nodes = edge_index[1].astype(i32)
    pad = E_PAD - N_EDGES
    dummy_dst = jnp.full((pad,), N_NODES, dtype=i32)
    zero_src = jnp.zeros((pad,), dtype=i32)

    tail = jnp.zeros((2 * E_CHUNK,), i32)
    ua_src = jnp.concatenate([i_nodes, zero_src, tail])
    ua_dst = jnp.concatenate([u_nodes, dummy_dst, tail])
    ia_src = jnp.concatenate([u_nodes, zero_src, tail])
    ia_dst = jnp.concatenate([i_nodes, dummy_dst, tail])
    zeros = jnp.zeros((N_PAD, HALF), jnp.float32)

    rpad = jnp.zeros((N_PAD - N_NODES, HALF), jnp.float32)
    ut_lo = jnp.concatenate([user_table[:, :HALF], rpad])
    ut_hi = jnp.concatenate([user_table[:, HALF:], rpad])
    it_lo = jnp.concatenate([item_table[:, :HALF], rpad])
    it_hi = jnp.concatenate([item_table[:, HALF:], rpad])
    sum_u_lo, sum_u_hi = ut_lo, ut_hi
    sum_i_lo, sum_i_hi = it_lo, it_hi
    ue_lo, ue_hi = ut_lo, ut_hi
    ie_lo, ie_hi = it_lo, it_hi

    ua_lo, ua_hi = _agg(ua_src, ua_dst, ie_lo, ie_hi, zeros)
    ia_lo, ia_hi = _agg(ia_src, ia_dst, ue_lo, ue_hi, zeros)
    for _ in range(2):
        ue_lo, ue_hi, sum_u_lo, sum_u_hi = _norm(ua_lo, ua_hi,
                                                 sum_u_lo, sum_u_hi)
        ia2_lo, ia2_hi = _agg(ia_src, ia_dst, ue_lo, ue_hi, zeros)
        ie_lo, ie_hi, sum_i_lo, sum_i_hi = _norm(ia_lo, ia_hi,
                                                 sum_i_lo, sum_i_hi)
        ua_lo, ua_hi = _agg(ua_src, ua_dst, ie_lo, ie_hi, zeros)
        ia_lo, ia_hi = ia2_lo, ia2_hi
    ue_lo, ue_hi, sum_u_lo, sum_u_hi = _norm(ua_lo, ua_hi,
                                             sum_u_lo, sum_u_hi)
    ie_lo, ie_hi, sum_i_lo, sum_i_hi = _norm(ia_lo, ia_hi,
                                             sum_i_lo, sum_i_hi)

    uidx = user_indices.astype(i32)
    iidx = item_indices.astype(i32)
    gul, guh, gil, gih = _batch_gather(sum_u_lo, sum_u_hi,
                                       sum_i_lo, sum_i_hi, uidx, iidx)
    return _dot(gul, guh, gil, gih).reshape(BATCH)
